# baseline (speedup 1.0000x reference)
import functools

import jax
import jax.numpy as jnp
from jax import lax
from jax.experimental import pallas as pl
from jax.experimental.pallas import tpu as pltpu

B = 4096
N = 50
NFEAT = 128
NHID = 64
NCLASS = 16

BB = 64


def _body(su_ref, sv_ref, sua_ref, sva_ref,
          w1_ref, b1_ref,
          tsu_ref, tsv_ref, f1b_ref,
          f2w_ref, f2b_ref, f3w_ref, f3b_ref, f4w_ref, f4b_ref,
          out_ref):
    bf = jnp.bfloat16
    w1 = w1_ref[...].astype(bf)
    b1 = b1_ref[...]

    def half(x2d, adj, t_ref):
        s1 = lax.dot_general(x2d, w1_ref[...], (((1,), (0,)), ((), ())),
                             preferred_element_type=jnp.float32)
        s1 = s1.reshape(BB, N, NHID)
        adj = adj.astype(bf)
        h1 = lax.dot_general(adj, s1.astype(bf), (((2,), (1,)), ((0,), (0,))),
                             preferred_element_type=jnp.float32)
        h1 = jnp.maximum(h1 + b1[None, None, :], 0.0)
        m2 = lax.dot_general(adj, h1.astype(bf), (((2,), (1,)), ((0,), (0,))),
                             preferred_element_type=jnp.float32)
        m2f = m2.reshape(BB, N * NHID)
        return lax.dot_general(m2f.astype(bf), t_ref[...],
                               (((1,), (0,)), ((), ())),
                               preferred_element_type=jnp.float32)

    d = half(su_ref[...], sua_ref[...], tsu_ref)
    d = d + half(sv_ref[...], sva_ref[...], tsv_ref)
    d = jnp.maximum(d + f1b_ref[...][None, :], 0.0)
    bfc = jnp.bfloat16
    d = jnp.maximum(
        jnp.dot(d.astype(bfc), f2w_ref[...].astype(bfc),
                preferred_element_type=jnp.float32)
        + f2b_ref[...][None, :], 0.0)
    d = jnp.maximum(
        jnp.dot(d.astype(bfc), f3w_ref[...].astype(bfc),
                preferred_element_type=jnp.float32)
        + f3b_ref[...][None, :], 0.0)
    d = (jnp.dot(d.astype(bfc), f4w_ref[...].astype(bfc),
                 preferred_element_type=jnp.float32)
         + f4b_ref[...][None, :])
    out_ref[...] = d


@jax.jit
def kernel(solute, solvent, solute_adj, solvent_adj,
           gc1_w, gc1_b, gc2_w, gc2_b,
           fc1_w, fc1_b, fc2_w, fc2_b, fc3_w, fc3_b, fc4_w, fc4_b):
    su2d = solute.reshape(B * N, NFEAT)
    sv2d = solvent.reshape(B * N, NFEAT)
    f3 = fc1_w.reshape(2 * N, NCLASS, 360)
    t_all = jnp.einsum('kc,ncf->nkf', gc2_w, f3)
    t_su = t_all[:N].reshape(N * NHID, 360).astype(jnp.bfloat16)
    t_sv = t_all[N:].reshape(N * NHID, 360).astype(jnp.bfloat16)
    f1b_eff = fc1_b + jnp.einsum('c,ncf->f', gc2_b, f3)

    grid = (B // BB,)

    def full_spec(arr):
        nd = arr.ndim
        return pl.BlockSpec(arr.shape, lambda i: (0,) * nd)

    in_specs = [
        pl.BlockSpec((BB * N, NFEAT), lambda i: (i, 0)),
        pl.BlockSpec((BB * N, NFEAT), lambda i: (i, 0)),
        pl.BlockSpec((BB, N, N), lambda i: (i, 0, 0)),
        pl.BlockSpec((BB, N, N), lambda i: (i, 0, 0)),
        full_spec(gc1_w), full_spec(gc1_b),
        full_spec(t_su), full_spec(t_sv), full_spec(f1b_eff),
        full_spec(fc2_w), full_spec(fc2_b),
        full_spec(fc3_w), full_spec(fc3_b),
        full_spec(fc4_w), full_spec(fc4_b),
    ]

    out = pl.pallas_call(
        _body,
        grid=grid,
        in_specs=in_specs,
        out_specs=pl.BlockSpec((BB, 1), lambda i: (i, 0)),
        out_shape=jax.ShapeDtypeStruct((B, 1), jnp.float32),
        compiler_params=pltpu.CompilerParams(
            dimension_semantics=("parallel",),
        ),
    )(su2d, sv2d, solute_adj, solvent_adj,
      gc1_w, gc1_b, t_su, t_sv, f1b_eff,
      fc2_w, fc2_b, fc3_w, fc3_b, fc4_w, fc4_b)
    return out


# speedup vs baseline: 1.0726x; 1.0726x over previous
import functools

import jax
import jax.numpy as jnp
from jax import lax
from jax.experimental import pallas as pl
from jax.experimental.pallas import tpu as pltpu

B = 4096
N = 50
NFEAT = 128
NHID = 64
NCLASS = 16

BB = 128


def _body(su_ref, sv_ref, sua_ref, sva_ref,
          w1_ref, b1_ref,
          tsu_ref, tsv_ref, f1b_ref,
          f2w_ref, f2b_ref, f3w_ref, f3b_ref, f4w_ref, f4b_ref,
          out_ref):
    bf = jnp.bfloat16
    w1 = w1_ref[...].astype(bf)
    b1 = b1_ref[...]

    def half(x2d, adj, t_ref):
        s1 = lax.dot_general(x2d, w1_ref[...], (((1,), (0,)), ((), ())),
                             preferred_element_type=jnp.float32)
        s1 = s1.reshape(BB, N, NHID)
        adj = adj.astype(bf)
        h1 = lax.dot_general(adj, s1.astype(bf), (((2,), (1,)), ((0,), (0,))),
                             preferred_element_type=jnp.float32)
        h1 = jnp.maximum(h1 + b1[None, None, :], 0.0)
        m2 = lax.dot_general(adj, h1.astype(bf), (((2,), (1,)), ((0,), (0,))),
                             preferred_element_type=jnp.float32)
        m2f = m2.reshape(BB, N * NHID)
        return lax.dot_general(m2f.astype(bf), t_ref[...],
                               (((1,), (0,)), ((), ())),
                               preferred_element_type=jnp.float32)

    d = half(su_ref[...], sua_ref[...], tsu_ref)
    d = d + half(sv_ref[...], sva_ref[...], tsv_ref)
    d = jnp.maximum(d + f1b_ref[...][None, :], 0.0)
    bfc = jnp.bfloat16
    d = jnp.maximum(
        jnp.dot(d.astype(bfc), f2w_ref[...].astype(bfc),
                preferred_element_type=jnp.float32)
        + f2b_ref[...][None, :], 0.0)
    d = jnp.maximum(
        jnp.dot(d.astype(bfc), f3w_ref[...].astype(bfc),
                preferred_element_type=jnp.float32)
        + f3b_ref[...][None, :], 0.0)
    d = (jnp.dot(d.astype(bfc), f4w_ref[...].astype(bfc),
                 preferred_element_type=jnp.float32)
         + f4b_ref[...][None, :])
    out_ref[...] = d


@jax.jit
def kernel(solute, solvent, solute_adj, solvent_adj,
           gc1_w, gc1_b, gc2_w, gc2_b,
           fc1_w, fc1_b, fc2_w, fc2_b, fc3_w, fc3_b, fc4_w, fc4_b):
    su2d = solute.reshape(B * N, NFEAT)
    sv2d = solvent.reshape(B * N, NFEAT)
    f3 = fc1_w.reshape(2 * N, NCLASS, 360)
    t_all = jnp.einsum('kc,ncf->nkf', gc2_w, f3)
    t_su = t_all[:N].reshape(N * NHID, 360).astype(jnp.bfloat16)
    t_sv = t_all[N:].reshape(N * NHID, 360).astype(jnp.bfloat16)
    f1b_eff = fc1_b + jnp.einsum('c,ncf->f', gc2_b, f3)

    grid = (B // BB,)

    def full_spec(arr):
        nd = arr.ndim
        return pl.BlockSpec(arr.shape, lambda i: (0,) * nd)

    in_specs = [
        pl.BlockSpec((BB * N, NFEAT), lambda i: (i, 0)),
        pl.BlockSpec((BB * N, NFEAT), lambda i: (i, 0)),
        pl.BlockSpec((BB, N, N), lambda i: (i, 0, 0)),
        pl.BlockSpec((BB, N, N), lambda i: (i, 0, 0)),
        full_spec(gc1_w), full_spec(gc1_b),
        full_spec(t_su), full_spec(t_sv), full_spec(f1b_eff),
        full_spec(fc2_w), full_spec(fc2_b),
        full_spec(fc3_w), full_spec(fc3_b),
        full_spec(fc4_w), full_spec(fc4_b),
    ]

    out = pl.pallas_call(
        _body,
        grid=grid,
        in_specs=in_specs,
        out_specs=pl.BlockSpec((BB, 1), lambda i: (i, 0)),
        out_shape=jax.ShapeDtypeStruct((B, 1), jnp.float32),
        compiler_params=pltpu.CompilerParams(
            dimension_semantics=("parallel",),
        ),
    )(su2d, sv2d, solute_adj, solvent_adj,
      gc1_w, gc1_b, t_su, t_sv, f1b_eff,
      fc2_w, fc2_b, fc3_w, fc3_b, fc4_w, fc4_b)
    return out


# speedup vs baseline: 1.3200x; 1.2306x over previous
import functools

import jax
import jax.numpy as jnp
from jax import lax
from jax.experimental import pallas as pl
from jax.experimental.pallas import tpu as pltpu

B = 4096
N = 50
NFEAT = 128
NHID = 64
NCLASS = 16

BB = 128


def _body(su_ref, sv_ref, sua_ref, sva_ref,
          w1_ref, b1_ref,
          tsu_ref, tsv_ref, f1b_ref,
          f2w_ref, f2b_ref, f3w_ref, f3b_ref, f4w_ref, f4b_ref,
          out_ref):
    bf = jnp.bfloat16
    w1 = w1_ref[...].astype(bf)
    b1 = b1_ref[...]

    def half(x3d, adj, t_ref):
        s1 = lax.dot_general(x3d.reshape(BB * N, NFEAT), w1_ref[...],
                             (((1,), (0,)), ((), ())),
                             preferred_element_type=jnp.float32)
        s1 = s1.reshape(BB, N, NHID)
        adj = adj.astype(bf)
        h1 = lax.dot_general(adj, s1.astype(bf), (((2,), (1,)), ((0,), (0,))),
                             preferred_element_type=jnp.float32)
        h1 = jnp.maximum(h1 + b1[None, None, :], 0.0)
        m2 = lax.dot_general(adj, h1.astype(bf), (((2,), (1,)), ((0,), (0,))),
                             preferred_element_type=jnp.float32)
        m2f = m2.reshape(BB, N * NHID)
        return lax.dot_general(m2f.astype(bf), t_ref[...],
                               (((1,), (0,)), ((), ())),
                               preferred_element_type=jnp.float32)

    d = half(su_ref[...], sua_ref[...], tsu_ref)
    d = d + half(sv_ref[...], sva_ref[...], tsv_ref)
    d = jnp.maximum(d + f1b_ref[...][None, :], 0.0)
    bfc = jnp.bfloat16
    d = jnp.maximum(
        jnp.dot(d.astype(bfc), f2w_ref[...].astype(bfc),
                preferred_element_type=jnp.float32)
        + f2b_ref[...][None, :], 0.0)
    d = jnp.maximum(
        jnp.dot(d.astype(bfc), f3w_ref[...].astype(bfc),
                preferred_element_type=jnp.float32)
        + f3b_ref[...][None, :], 0.0)
    d = (jnp.dot(d.astype(bfc), f4w_ref[...].astype(bfc),
                 preferred_element_type=jnp.float32)
         + f4b_ref[...][None, :])
    out_ref[...] = d


@jax.jit
def kernel(solute, solvent, solute_adj, solvent_adj,
           gc1_w, gc1_b, gc2_w, gc2_b,
           fc1_w, fc1_b, fc2_w, fc2_b, fc3_w, fc3_b, fc4_w, fc4_b):
    f3 = fc1_w.reshape(2 * N, NCLASS, 360)
    t_all = jnp.einsum('kc,ncf->nkf', gc2_w, f3)
    t_su = t_all[:N].reshape(N * NHID, 360).astype(jnp.bfloat16)
    t_sv = t_all[N:].reshape(N * NHID, 360).astype(jnp.bfloat16)
    f1b_eff = fc1_b + jnp.einsum('c,ncf->f', gc2_b, f3)

    grid = (B // BB,)

    def full_spec(arr):
        nd = arr.ndim
        return pl.BlockSpec(arr.shape, lambda i: (0,) * nd)

    in_specs = [
        pl.BlockSpec((BB, N, NFEAT), lambda i: (i, 0, 0)),
        pl.BlockSpec((BB, N, NFEAT), lambda i: (i, 0, 0)),
        pl.BlockSpec((BB, N, N), lambda i: (i, 0, 0)),
        pl.BlockSpec((BB, N, N), lambda i: (i, 0, 0)),
        full_spec(gc1_w), full_spec(gc1_b),
        full_spec(t_su), full_spec(t_sv), full_spec(f1b_eff),
        full_spec(fc2_w), full_spec(fc2_b),
        full_spec(fc3_w), full_spec(fc3_b),
        full_spec(fc4_w), full_spec(fc4_b),
    ]

    out = pl.pallas_call(
        _body,
        grid=grid,
        in_specs=in_specs,
        out_specs=pl.BlockSpec((BB, 1), lambda i: (i, 0)),
        out_shape=jax.ShapeDtypeStruct((B, 1), jnp.float32),
        compiler_params=pltpu.CompilerParams(
            dimension_semantics=("parallel",),
        ),
    )(solute, solvent, solute_adj, solvent_adj,
      gc1_w, gc1_b, t_su, t_sv, f1b_eff,
      fc2_w, fc2_b, fc3_w, fc3_b, fc4_w, fc4_b)
    return out
